# baseline (speedup 1.0000x reference)
import jax
import jax.numpy as jnp
from jax import lax
from jax.experimental import pallas as pl
from jax.experimental.pallas import tpu as pltpu
from jax.experimental.pallas import tpu_sc as plsc

N = 10000
E = 320000
D = 128
NG = 64
EPS = 1e-5

_SC_CORES = 2
_SC_TILES = 16
_NW = _SC_CORES * _SC_TILES
_EC = 125
_ECH = 80
_QC = 96
_NQC = (E // _NW) // _QC
_QR = (E // _NW) - _NQC * _QC
_U = 4
_NQG = _NQC // _U
_RCH = 80
_NECH = N // _RCH


def _mesh():
    return plsc.VectorSubcoreMesh(core_axis_name="c", subcore_axis_name="s",
                                  num_cores=_SC_CORES, num_subcores=_SC_TILES)


def _embed_deg_body(nf_hbm, tab_hbm, src3_hbm, dst3_hbm,
                    h0_hbm, dego_hbm, degi_hbm,
                    sAll, dAll, ones_v, zv, idx_e, row_e,
                    dego_sh, degi_sh, semD, semE):
    c = lax.axis_index("c")
    s = lax.axis_index("s")
    w = c * _SC_TILES + s

    for k in range(8):
        ones_v[pl.ds(16 * k, 16)] = jnp.ones((16,), jnp.float32)
    for k in range(40):
        zv[pl.ds(16 * k, 16)] = jnp.zeros((16,), jnp.float32)

    @pl.when(s < _SC_TILES - 1)
    def _():
        pltpu.sync_copy(zv.at[pl.ds(0, 624)], dego_sh.at[pl.ds(s * 624, 624)])
        pltpu.sync_copy(zv.at[pl.ds(0, 624)], degi_sh.at[pl.ds(s * 624, 624)])

    @pl.when(s == _SC_TILES - 1)
    def _():
        pltpu.sync_copy(zv, dego_sh.at[pl.ds((_SC_TILES - 1) * 624, 640)])
        pltpu.sync_copy(zv, degi_sh.at[pl.ds((_SC_TILES - 1) * 624, 640)])

    pltpu.sync_copy(src3_hbm.at[pl.ds(w * _ECH, _ECH)], sAll)
    pltpu.sync_copy(dst3_hbm.at[pl.ds(w * _ECH, _ECH)], dAll)

    for t in range(4):
        ch = w + _NW * t

        @pl.when(ch < _NECH)
        def _():
            pltpu.sync_copy(nf_hbm.at[pl.ds(ch * _RCH, _RCH)], idx_e[t])
            pltpu.async_copy(tab_hbm.at[idx_e[t]], row_e[t], semE[t])

    plsc.subcore_barrier()

    def deg_body(g, carry):
        ones = ones_v.at[pl.ds(0, _EC)]
        for p in range(8):
            ch = g * 8 + p
            pltpu.async_copy(ones, dego_sh.at[sAll.at[ch]], semD[0], add=True)
            pltpu.async_copy(ones, degi_sh.at[dAll.at[ch]], semD[1], add=True)
        for p in range(8):
            ch = g * 8 + p
            pltpu.make_async_copy(ones, dego_sh.at[sAll.at[ch]], semD[0]).wait()
            pltpu.make_async_copy(ones, degi_sh.at[dAll.at[ch]], semD[1]).wait()
        return carry

    lax.fori_loop(0, _ECH // 8, deg_body, 0)

    for t in range(4):
        ch = w + _NW * t

        @pl.when(ch < _NECH)
        def _():
            pltpu.make_async_copy(tab_hbm.at[idx_e[t]], row_e[t], semE[t]).wait()
            pltpu.sync_copy(row_e[t], h0_hbm.at[pl.ds(ch * _RCH, _RCH)])

    plsc.subcore_barrier()

    @pl.when(s == 0)
    def _():
        pltpu.sync_copy(dego_sh, dego_hbm.at[c])
        pltpu.sync_copy(degi_sh, degi_hbm.at[c])


def _sc_embed_deg(nf, table, src3, dst3):
    f = pl.kernel(
        _embed_deg_body,
        out_type=(jax.ShapeDtypeStruct((N, D), jnp.float32),
                  jax.ShapeDtypeStruct((_SC_CORES, N), jnp.float32),
                  jax.ShapeDtypeStruct((_SC_CORES, N), jnp.float32)),
        mesh=_mesh(),
        scratch_types=[
            pltpu.VMEM((_ECH, _EC), jnp.int32),
            pltpu.VMEM((_ECH, _EC), jnp.int32),
            pltpu.VMEM((128,), jnp.float32),
            pltpu.VMEM((640,), jnp.float32),
            [pltpu.VMEM((_RCH,), jnp.int32) for _ in range(4)],
            [pltpu.VMEM((_RCH, D), jnp.float32) for _ in range(4)],
            pltpu.VMEM_SHARED((N,), jnp.float32),
            pltpu.VMEM_SHARED((N,), jnp.float32),
            [pltpu.SemaphoreType.DMA for _ in range(2)],
            [pltpu.SemaphoreType.DMA for _ in range(4)],
        ],
    )
    return f(nf, table, src3, dst3)


def _agg_body(hs_hbm, src_hbm, dst_hbm, zero_hbm, out_hbm,
              sI, dI, sR, dR, gbuf, agg_sh, semG, semS):
    c = lax.axis_index("c")
    s = lax.axis_index("s")
    w = c * _SC_TILES + s
    ebase = w * (E // _NW)

    @pl.when(s < _SC_TILES - 1)
    def _():
        pltpu.sync_copy(zero_hbm.at[pl.ds(s * 624, 624)],
                        agg_sh.at[pl.ds(s * 624, 624)])

    @pl.when(s == _SC_TILES - 1)
    def _():
        pltpu.sync_copy(zero_hbm.at[pl.ds((_SC_TILES - 1) * 624, 640)],
                        agg_sh.at[pl.ds((_SC_TILES - 1) * 624, 640)])

    plsc.subcore_barrier()

    def load_idx(ch, si, di):
        off = ebase + ch * _QC
        pltpu.sync_copy(src_hbm.at[pl.ds(off, _QC)], si)
        pltpu.sync_copy(dst_hbm.at[pl.ds(off, _QC)], di)

    for p in range(_U):
        load_idx(p, sI[p], dI[p])
        pltpu.async_copy(hs_hbm.at[pl.ds(p * _QC, _QC)], gbuf[p], semG[p])

    def body(i, carry):
        for p in range(_U):
            pltpu.make_async_copy(hs_hbm.at[pl.ds((i * _U + p) % 64 * _QC, _QC)], gbuf[p], semG[p]).wait()
            pltpu.async_copy(gbuf[p], agg_sh.at[dI[p]], semS[p], add=True)
        for p in range(_U):
            pltpu.make_async_copy(gbuf[p], agg_sh.at[dI[p]], semS[p]).wait()

            @pl.when(i < _NQG - 1)
            def _():
                load_idx((i + 1) * _U + p, sI[p], dI[p])
                pltpu.async_copy(hs_hbm.at[pl.ds((i * _U + _U + p) % 64 * _QC, _QC)], gbuf[p], semG[p])
        return carry

    lax.fori_loop(0, _NQG, body, 0)

    roff = ebase + _NQC * _QC
    pltpu.sync_copy(src_hbm.at[pl.ds(roff, _QR)], sR)
    pltpu.sync_copy(dst_hbm.at[pl.ds(roff, _QR)], dR)
    pltpu.async_copy(hs_hbm.at[sR], gbuf[0].at[pl.ds(0, _QR)], semG[0]).wait()
    pltpu.sync_copy(gbuf[0].at[pl.ds(0, _QR)], agg_sh.at[dR], add=True)

    plsc.subcore_barrier()

    @pl.when(s < _SC_TILES - 1)
    def _():
        pltpu.sync_copy(agg_sh.at[pl.ds(s * 624, 624)],
                        out_hbm.at[c, pl.ds(s * 624, 624)])

    @pl.when(s == _SC_TILES - 1)
    def _():
        pltpu.sync_copy(agg_sh.at[pl.ds((_SC_TILES - 1) * 624, 640)],
                        out_hbm.at[c, pl.ds((_SC_TILES - 1) * 624, 640)])


def _sc_aggregate(hs, src, dst, zeros_h):
    f = pl.kernel(
        _agg_body,
        out_type=jax.ShapeDtypeStruct((_SC_CORES, N, D), jnp.float32),
        mesh=_mesh(),
        scratch_types=[
            [pltpu.VMEM((_QC,), jnp.int32) for _ in range(_U)],
            [pltpu.VMEM((_QC,), jnp.int32) for _ in range(_U)],
            pltpu.VMEM((_QR,), jnp.int32),
            pltpu.VMEM((_QR,), jnp.int32),
            [pltpu.VMEM((_QC, D), jnp.float32) for _ in range(_U)],
            pltpu.VMEM_SHARED((N, D), jnp.float32),
            [pltpu.SemaphoreType.DMA for _ in range(_U)],
            [pltpu.SemaphoreType.DMA for _ in range(_U)],
        ],
    )
    return f(hs, src, dst, zeros_h)


def _prep_body(h0_ref, degT_ref, hs_ref, dd_ref):
    degT = degT_ref[...]
    deg_o = jnp.maximum(degT[:, 0:1] + degT[:, 1:2], 1.0)
    deg_i = jnp.maximum(degT[:, 2:3] + degT[:, 3:4], 1.0)
    dout = lax.rsqrt(deg_o)
    din = lax.rsqrt(deg_i)
    hs_ref[...] = h0_ref[...] * dout
    dd_ref[...] = jnp.concatenate([din, dout], axis=1)


def _bn_relu(y, g, be):
    m = jnp.mean(y, axis=0, keepdims=True)
    v = jnp.mean((y - m) ** 2, axis=0, keepdims=True)
    y = (y - m) * lax.rsqrt(v + EPS) * g + be
    return jnp.maximum(y, 0.0)


def _layer_body(agg_ref, dd_ref, W_ref, b_ref, g_ref, be_ref, hs_ref):
    x = agg_ref[0] + agg_ref[1]
    dd = dd_ref[...]
    y = dd[:, 0:1] * jnp.dot(x, W_ref[...], preferred_element_type=jnp.float32) \
        + b_ref[...]
    y = _bn_relu(y, g_ref[...], be_ref[...])
    hs_ref[...] = y * dd[:, 1:2]


def _final_body(agg_ref, dd_ref, gid_ref, W_ref, b_ref, g_ref, be_ref,
                fc1W_ref, fc1b_ref, fc2W_ref, fc2b_ref, out_ref):
    x = agg_ref[0] + agg_ref[1]
    dd = dd_ref[...]
    y = dd[:, 0:1] * jnp.dot(x, W_ref[...], preferred_element_type=jnp.float32) \
        + b_ref[...]
    y = _bn_relu(y, g_ref[...], be_ref[...])
    gid = gid_ref[...]
    seg = lax.broadcasted_iota(jnp.int32, (NG, N), 0)
    maskT = (seg == gid).astype(jnp.float32)
    counts = jnp.sum(maskT, axis=1, keepdims=True)
    hg = jnp.dot(maskT, y, preferred_element_type=jnp.float32)
    hg = hg / jnp.maximum(counts, 1.0)
    z = jnp.maximum(
        jnp.dot(hg, fc1W_ref[...], preferred_element_type=jnp.float32)
        + fc1b_ref[...], 0.0)
    out_ref[...] = jnp.dot(z, fc2W_ref[...],
                           preferred_element_type=jnp.float32) + fc2b_ref[...]


def kernel(node_feat, edge_index, graph_ids, embed_table, W1, b1, g1, be1,
           W2, b2, g2, be2, W3, b3, g3, be3, fc1W, fc1b, fc2W, fc2b):
    nf = node_feat[:, 0]
    src = edge_index[0]
    dst = edge_index[1]
    src3 = src.reshape(E // _EC, _EC)
    dst3 = dst.reshape(E // _EC, _EC)

    h0, dego, degi = _sc_embed_deg(nf, embed_table, src3, dst3)
    degT = jnp.stack([dego[0], dego[1], degi[0], degi[1]], axis=1)

    hs, dd = pl.pallas_call(
        _prep_body,
        out_shape=(jax.ShapeDtypeStruct((N, D), jnp.float32),
                   jax.ShapeDtypeStruct((N, 2), jnp.float32)),
    )(h0, degT)

    zeros_h = jnp.zeros((N, D), jnp.float32)

    for (W, b, g, be) in ((W1, b1, g1, be1), (W2, b2, g2, be2)):
        agg = _sc_aggregate(hs, src, dst, zeros_h)
        hs = pl.pallas_call(
            _layer_body,
            out_shape=jax.ShapeDtypeStruct((N, D), jnp.float32),
        )(agg, dd, W, b.reshape(1, D), g.reshape(1, D), be.reshape(1, D))

    agg = _sc_aggregate(hs, src, dst, zeros_h)
    out = pl.pallas_call(
        _final_body,
        out_shape=jax.ShapeDtypeStruct((NG, fc2W.shape[1]), jnp.float32),
    )(agg, dd, graph_ids.reshape(1, N), W3, b3.reshape(1, D),
      g3.reshape(1, D), be3.reshape(1, D),
      fc1W, fc1b.reshape(1, -1), fc2W, fc2b.reshape(1, -1))
    return out


# speedup vs baseline: 1.0603x; 1.0603x over previous
import jax
import jax.numpy as jnp
from jax import lax
from jax.experimental import pallas as pl
from jax.experimental.pallas import tpu as pltpu
from jax.experimental.pallas import tpu_sc as plsc

N = 10000
E = 320000
D = 128
NG = 64
EPS = 1e-5

_SC_CORES = 2
_SC_TILES = 16
_NW = _SC_CORES * _SC_TILES
_EC = 125
_ECH = 80
_QC = 96
_NQC = (E // _NW) // _QC
_QR = (E // _NW) - _NQC * _QC
_U = 4
_NQG = _NQC // _U
_RCH = 80
_NECH = N // _RCH


def _mesh():
    return plsc.VectorSubcoreMesh(core_axis_name="c", subcore_axis_name="s",
                                  num_cores=_SC_CORES, num_subcores=_SC_TILES)


def _embed_deg_body(nf_hbm, tab_hbm, src3_hbm, dst3_hbm,
                    h0_hbm, dego_hbm, degi_hbm,
                    sAll, dAll, ones_v, zv, idx_e, row_e,
                    dego_sh, degi_sh, semD, semE):
    c = lax.axis_index("c")
    s = lax.axis_index("s")
    w = c * _SC_TILES + s

    for k in range(8):
        ones_v[pl.ds(16 * k, 16)] = jnp.ones((16,), jnp.float32)
    for k in range(40):
        zv[pl.ds(16 * k, 16)] = jnp.zeros((16,), jnp.float32)

    @pl.when(s < _SC_TILES - 1)
    def _():
        pltpu.sync_copy(zv.at[pl.ds(0, 624)], dego_sh.at[pl.ds(s * 624, 624)])
        pltpu.sync_copy(zv.at[pl.ds(0, 624)], degi_sh.at[pl.ds(s * 624, 624)])

    @pl.when(s == _SC_TILES - 1)
    def _():
        pltpu.sync_copy(zv, dego_sh.at[pl.ds((_SC_TILES - 1) * 624, 640)])
        pltpu.sync_copy(zv, degi_sh.at[pl.ds((_SC_TILES - 1) * 624, 640)])

    pltpu.sync_copy(src3_hbm.at[pl.ds(w * _ECH, _ECH)], sAll)
    pltpu.sync_copy(dst3_hbm.at[pl.ds(w * _ECH, _ECH)], dAll)

    for t in range(4):
        ch = w + _NW * t

        @pl.when(ch < _NECH)
        def _():
            pltpu.sync_copy(nf_hbm.at[pl.ds(ch * _RCH, _RCH)], idx_e[t])
            pltpu.async_copy(tab_hbm.at[idx_e[t]], row_e[t], semE[t])

    plsc.subcore_barrier()

    def deg_body(g, carry):
        ones = ones_v.at[pl.ds(0, _EC)]
        for p in range(8):
            ch = g * 8 + p
            pltpu.async_copy(ones, dego_sh.at[sAll.at[ch]], semD[0], add=True)
            pltpu.async_copy(ones, degi_sh.at[dAll.at[ch]], semD[1], add=True)
        for p in range(8):
            ch = g * 8 + p
            pltpu.make_async_copy(ones, dego_sh.at[sAll.at[ch]], semD[0]).wait()
            pltpu.make_async_copy(ones, degi_sh.at[dAll.at[ch]], semD[1]).wait()
        return carry

    lax.fori_loop(0, _ECH // 8, deg_body, 0)

    for t in range(4):
        ch = w + _NW * t

        @pl.when(ch < _NECH)
        def _():
            pltpu.make_async_copy(tab_hbm.at[idx_e[t]], row_e[t], semE[t]).wait()
            pltpu.sync_copy(row_e[t], h0_hbm.at[pl.ds(ch * _RCH, _RCH)])

    plsc.subcore_barrier()

    @pl.when(s == 0)
    def _():
        pltpu.sync_copy(dego_sh, dego_hbm.at[c])
        pltpu.sync_copy(degi_sh, degi_hbm.at[c])


def _sc_embed_deg(nf, table, src3, dst3):
    f = pl.kernel(
        _embed_deg_body,
        out_type=(jax.ShapeDtypeStruct((N, D), jnp.float32),
                  jax.ShapeDtypeStruct((_SC_CORES, N), jnp.float32),
                  jax.ShapeDtypeStruct((_SC_CORES, N), jnp.float32)),
        mesh=_mesh(),
        scratch_types=[
            pltpu.VMEM((_ECH, _EC), jnp.int32),
            pltpu.VMEM((_ECH, _EC), jnp.int32),
            pltpu.VMEM((128,), jnp.float32),
            pltpu.VMEM((640,), jnp.float32),
            [pltpu.VMEM((_RCH,), jnp.int32) for _ in range(4)],
            [pltpu.VMEM((_RCH, D), jnp.float32) for _ in range(4)],
            pltpu.VMEM_SHARED((N,), jnp.float32),
            pltpu.VMEM_SHARED((N,), jnp.float32),
            [pltpu.SemaphoreType.DMA for _ in range(2)],
            [pltpu.SemaphoreType.DMA for _ in range(4)],
        ],
    )
    return f(nf, table, src3, dst3)


def _agg_body(hs_hbm, src_hbm, dst_hbm, out_hbm,
              sI, dI, sR, dR, gbuf, agg_sh, semG, semS):
    c = lax.axis_index("c")
    s = lax.axis_index("s")
    w = c * _SC_TILES + s
    ebase = w * (E // _NW)

    def zrow(i, carry):
        for k in range(8):
            gbuf[0][i, pl.ds(16 * k, 16)] = jnp.zeros((16,), jnp.float32)
        return carry

    lax.fori_loop(0, _QC, zrow, 0)
    for q in range(6):
        pltpu.sync_copy(gbuf[0], agg_sh.at[pl.ds(s * 624 + q * _QC, _QC)])
    pltpu.sync_copy(gbuf[0].at[pl.ds(0, 48)],
                    agg_sh.at[pl.ds(s * 624 + 576, 48)])

    @pl.when(s == _SC_TILES - 1)
    def _():
        pltpu.sync_copy(gbuf[0].at[pl.ds(0, 16)],
                        agg_sh.at[pl.ds(_SC_TILES * 624, 16)])

    plsc.subcore_barrier()

    def load_idx(ch, si, di):
        off = ebase + ch * _QC
        pltpu.sync_copy(src_hbm.at[pl.ds(off, _QC)], si)
        pltpu.sync_copy(dst_hbm.at[pl.ds(off, _QC)], di)

    for p in range(_U):
        load_idx(p, sI[p], dI[p])
        pltpu.async_copy(hs_hbm.at[sI[p]], gbuf[p], semG[p])

    def body(i, carry):
        for p in range(_U):
            pltpu.make_async_copy(hs_hbm.at[sI[p]], gbuf[p], semG[p]).wait()
            pltpu.async_copy(gbuf[p], agg_sh.at[dI[p]], semS[p], add=True)
        for p in range(_U):
            pltpu.make_async_copy(gbuf[p], agg_sh.at[dI[p]], semS[p]).wait()

            @pl.when(i < _NQG - 1)
            def _():
                load_idx((i + 1) * _U + p, sI[p], dI[p])
                pltpu.async_copy(hs_hbm.at[sI[p]], gbuf[p], semG[p])
        return carry

    lax.fori_loop(0, _NQG, body, 0)

    roff = ebase + _NQC * _QC
    pltpu.sync_copy(src_hbm.at[pl.ds(roff, _QR)], sR)
    pltpu.sync_copy(dst_hbm.at[pl.ds(roff, _QR)], dR)
    pltpu.async_copy(hs_hbm.at[sR], gbuf[0].at[pl.ds(0, _QR)], semG[0]).wait()
    pltpu.sync_copy(gbuf[0].at[pl.ds(0, _QR)], agg_sh.at[dR], add=True)

    plsc.subcore_barrier()

    @pl.when(s < _SC_TILES - 1)
    def _():
        pltpu.sync_copy(agg_sh.at[pl.ds(s * 624, 624)],
                        out_hbm.at[c, pl.ds(s * 624, 624)])

    @pl.when(s == _SC_TILES - 1)
    def _():
        pltpu.sync_copy(agg_sh.at[pl.ds((_SC_TILES - 1) * 624, 640)],
                        out_hbm.at[c, pl.ds((_SC_TILES - 1) * 624, 640)])


def _sc_aggregate(hs, src, dst):
    f = pl.kernel(
        _agg_body,
        out_type=jax.ShapeDtypeStruct((_SC_CORES, N, D), jnp.float32),
        mesh=_mesh(),
        scratch_types=[
            [pltpu.VMEM((_QC,), jnp.int32) for _ in range(_U)],
            [pltpu.VMEM((_QC,), jnp.int32) for _ in range(_U)],
            pltpu.VMEM((_QR,), jnp.int32),
            pltpu.VMEM((_QR,), jnp.int32),
            [pltpu.VMEM((_QC, D), jnp.float32) for _ in range(_U)],
            pltpu.VMEM_SHARED((N, D), jnp.float32),
            [pltpu.SemaphoreType.DMA for _ in range(_U)],
            [pltpu.SemaphoreType.DMA for _ in range(_U)],
        ],
    )
    return f(hs, src, dst)


def _prep_body(h0_ref, degT_ref, hs_ref, dd_ref):
    degT = degT_ref[...]
    deg_o = jnp.maximum(degT[:, 0:1] + degT[:, 1:2], 1.0)
    deg_i = jnp.maximum(degT[:, 2:3] + degT[:, 3:4], 1.0)
    dout = lax.rsqrt(deg_o)
    din = lax.rsqrt(deg_i)
    hs_ref[...] = h0_ref[...] * dout
    dd_ref[...] = jnp.concatenate([din, dout], axis=1)


def _bn_relu(y, g, be):
    m = jnp.mean(y, axis=0, keepdims=True)
    v = jnp.mean((y - m) ** 2, axis=0, keepdims=True)
    y = (y - m) * lax.rsqrt(v + EPS) * g + be
    return jnp.maximum(y, 0.0)


def _layer_body(agg_ref, dd_ref, W_ref, b_ref, g_ref, be_ref, hs_ref):
    x = agg_ref[0] + agg_ref[1]
    dd = dd_ref[...]
    y = dd[:, 0:1] * jnp.dot(x, W_ref[...], preferred_element_type=jnp.float32) \
        + b_ref[...]
    y = _bn_relu(y, g_ref[...], be_ref[...])
    hs_ref[...] = y * dd[:, 1:2]


def _final_body(agg_ref, dd_ref, gid_ref, W_ref, b_ref, g_ref, be_ref,
                fc1W_ref, fc1b_ref, fc2W_ref, fc2b_ref, out_ref):
    x = agg_ref[0] + agg_ref[1]
    dd = dd_ref[...]
    y = dd[:, 0:1] * jnp.dot(x, W_ref[...], preferred_element_type=jnp.float32) \
        + b_ref[...]
    y = _bn_relu(y, g_ref[...], be_ref[...])
    gid = gid_ref[...]
    seg = lax.broadcasted_iota(jnp.int32, (NG, N), 0)
    maskT = (seg == gid).astype(jnp.float32)
    counts = jnp.sum(maskT, axis=1, keepdims=True)
    hg = jnp.dot(maskT, y, preferred_element_type=jnp.float32)
    hg = hg / jnp.maximum(counts, 1.0)
    z = jnp.maximum(
        jnp.dot(hg, fc1W_ref[...], preferred_element_type=jnp.float32)
        + fc1b_ref[...], 0.0)
    out_ref[...] = jnp.dot(z, fc2W_ref[...],
                           preferred_element_type=jnp.float32) + fc2b_ref[...]


def kernel(node_feat, edge_index, graph_ids, embed_table, W1, b1, g1, be1,
           W2, b2, g2, be2, W3, b3, g3, be3, fc1W, fc1b, fc2W, fc2b):
    nf = node_feat[:, 0]
    src = edge_index[0]
    dst = edge_index[1]
    src3 = src.reshape(E // _EC, _EC)
    dst3 = dst.reshape(E // _EC, _EC)

    h0, dego, degi = _sc_embed_deg(nf, embed_table, src3, dst3)
    degT = jnp.stack([dego[0], dego[1], degi[0], degi[1]], axis=1)

    hs, dd = pl.pallas_call(
        _prep_body,
        out_shape=(jax.ShapeDtypeStruct((N, D), jnp.float32),
                   jax.ShapeDtypeStruct((N, 2), jnp.float32)),
    )(h0, degT)

    for (W, b, g, be) in ((W1, b1, g1, be1), (W2, b2, g2, be2)):
        agg = _sc_aggregate(hs, src, dst)
        hs = pl.pallas_call(
            _layer_body,
            out_shape=jax.ShapeDtypeStruct((N, D), jnp.float32),
        )(agg, dd, W, b.reshape(1, D), g.reshape(1, D), be.reshape(1, D))

    agg = _sc_aggregate(hs, src, dst)
    out = pl.pallas_call(
        _final_body,
        out_shape=jax.ShapeDtypeStruct((NG, fc2W.shape[1]), jnp.float32),
    )(agg, dd, graph_ids.reshape(1, N), W3, b3.reshape(1, D),
      g3.reshape(1, D), be3.reshape(1, D),
      fc1W, fc1b.reshape(1, -1), fc2W, fc2b.reshape(1, -1))
    return out
